# baseline (speedup 1.0000x reference)
import jax
import jax.numpy as jnp
from jax.experimental import pallas as pl
from jax.experimental.pallas import tpu as pltpu


def _xw_kernel(x_ref, w_ref, o_ref):
    o_ref[...] = jnp.dot(
        x_ref[...].astype(jnp.bfloat16), w_ref[...],
        preferred_element_type=jnp.float32).astype(jnp.bfloat16)


def _layer1_kernel(a_ref, z_ref, b1_ref, w2_ref, o_ref):
    h = jnp.dot(a_ref[...], z_ref[...], preferred_element_type=jnp.float32)
    h = h + b1_ref[...]
    h = jnp.where(h > 0, h, 0.2 * h)
    o_ref[...] = jnp.dot(
        h.astype(jnp.bfloat16), w2_ref[...],
        preferred_element_type=jnp.float32).astype(jnp.bfloat16)


def _layer2_kernel(a_ref, u_ref, b2_ref, o_ref):
    y = jnp.dot(a_ref[...], u_ref[...], preferred_element_type=jnp.float32)
    y = y + b2_ref[...]
    m = jnp.max(y, axis=1, keepdims=True)
    e = jnp.exp(y - m)
    o_ref[...] = y - (m + jnp.log(jnp.sum(e, axis=1, keepdims=True)))


def _build_a_hat(edge_index, n):
    src = edge_index[0].astype(jnp.int32)
    dst = edge_index[1].astype(jnp.int32)
    ones = jnp.ones(src.shape, jnp.float32)
    indeg = jnp.zeros((n,), jnp.float32).at[dst].add(ones)
    self_cnt = jnp.zeros((n,), jnp.float32).at[dst].add(
        jnp.where(src == dst, 1.0, 0.0))
    no_self = self_cnt == 0.0
    deg = indeg + jnp.where(no_self, 1.0, 0.0)
    d = jax.lax.rsqrt(deg)
    counts = jnp.zeros((n, n), jnp.bfloat16).at[dst, src].add(
        jnp.ones(src.shape, jnp.bfloat16))
    i_idx = jax.lax.broadcasted_iota(jnp.int32, (n, n), 0)
    j_idx = jax.lax.broadcasted_iota(jnp.int32, (n, n), 1)
    diag_add = jnp.where((i_idx == j_idx) & no_self[:, None], 1.0, 0.0)
    a = (counts.astype(jnp.float32) + diag_add) * d[:, None] * d[None, :]
    return a.astype(jnp.bfloat16)


def kernel(x, edge_index, w1, b1, w2, b2):
    n, fin = x.shape
    hidden = w1.shape[1]
    c = w2.shape[1]
    tm = 512 if n % 512 == 0 else n
    grid = (n // tm,)

    a = _build_a_hat(edge_index, n)
    w1b = w1.astype(jnp.bfloat16)
    w2b = w2.astype(jnp.bfloat16)
    b1f = b1.reshape(1, hidden).astype(jnp.float32)
    b2f = b2.reshape(1, c).astype(jnp.float32)

    z = pl.pallas_call(
        _xw_kernel,
        out_shape=jax.ShapeDtypeStruct((n, hidden), jnp.bfloat16),
        grid=grid,
        in_specs=[pl.BlockSpec((tm, fin), lambda i: (i, 0)),
                  pl.BlockSpec((fin, hidden), lambda i: (0, 0))],
        out_specs=pl.BlockSpec((tm, hidden), lambda i: (i, 0)),
        compiler_params=pltpu.CompilerParams(
            dimension_semantics=("parallel",),
            vmem_limit_bytes=48 << 20,
        ),
        cost_estimate=pl.CostEstimate(
            flops=2 * n * fin * hidden, transcendentals=0,
            bytes_accessed=n * fin * 4 + fin * hidden * 2 + n * hidden * 2),
    )(x, w1b)

    u = pl.pallas_call(
        _layer1_kernel,
        out_shape=jax.ShapeDtypeStruct((n, c), jnp.bfloat16),
        grid=grid,
        in_specs=[pl.BlockSpec((tm, n), lambda i: (i, 0)),
                  pl.BlockSpec((n, hidden), lambda i: (0, 0)),
                  pl.BlockSpec((1, hidden), lambda i: (0, 0)),
                  pl.BlockSpec((hidden, c), lambda i: (0, 0))],
        out_specs=pl.BlockSpec((tm, c), lambda i: (i, 0)),
        compiler_params=pltpu.CompilerParams(
            dimension_semantics=("parallel",),
            vmem_limit_bytes=48 << 20,
        ),
        cost_estimate=pl.CostEstimate(
            flops=2 * n * n * hidden + 2 * n * hidden * c, transcendentals=0,
            bytes_accessed=n * n * 2 + n * hidden * 2 + n * c * 2),
    )(a, z, b1f, w2b)

    out = pl.pallas_call(
        _layer2_kernel,
        out_shape=jax.ShapeDtypeStruct((n, c), jnp.float32),
        grid=grid,
        in_specs=[pl.BlockSpec((tm, n), lambda i: (i, 0)),
                  pl.BlockSpec((n, c), lambda i: (0, 0)),
                  pl.BlockSpec((1, c), lambda i: (0, 0))],
        out_specs=pl.BlockSpec((tm, c), lambda i: (i, 0)),
        compiler_params=pltpu.CompilerParams(
            dimension_semantics=("parallel",),
            vmem_limit_bytes=48 << 20,
        ),
        cost_estimate=pl.CostEstimate(
            flops=2 * n * n * c, transcendentals=n * c + n,
            bytes_accessed=n * n * 2 + n * c * 2 + n * c * 4),
    )(a, u, b2f)

    return out


# speedup vs baseline: 1.1338x; 1.1338x over previous
import functools

import jax
import jax.numpy as jnp
from jax.experimental import pallas as pl
from jax.experimental.pallas import tpu as pltpu


def _tile_norm(c_ref, tm):
    i = pl.program_id(0)
    deg = jnp.sum(c_ref[...].astype(jnp.float32), axis=1, keepdims=True)
    csub = c_ref[:, pl.ds(pl.multiple_of(i * tm, tm), tm)].astype(jnp.float32)
    r = jax.lax.broadcasted_iota(jnp.int32, (tm, tm), 0)
    col = jax.lax.broadcasted_iota(jnp.int32, (tm, tm), 1)
    self_cnt = jnp.sum(jnp.where(r == col, csub, 0.0), axis=1, keepdims=True)
    no_self = (self_cnt == 0.0).astype(jnp.float32)
    d = jax.lax.rsqrt(deg + no_self)
    return d, no_self


def _prep_kernel(x_ref, w1_ref, c_ref, z_ref, d_ref, *, tm):
    d, _ = _tile_norm(c_ref, tm)
    z_ref[...] = jnp.dot(x_ref[...].astype(jnp.bfloat16), w1_ref[...],
                         preferred_element_type=jnp.float32
                         ).astype(jnp.bfloat16)
    d_ref[...] = d.reshape(1, tm)


def _scaled_dot(c_ref, dall_ref, v_ref, tm):
    i = pl.program_id(0)
    d, no_self = _tile_norm(c_ref, tm)
    a = ((c_ref[...].astype(jnp.float32) * d) * dall_ref[...]
         ).astype(jnp.bfloat16)
    acc = jnp.dot(a, v_ref[...], preferred_element_type=jnp.float32)
    vrows = v_ref[pl.ds(pl.multiple_of(i * tm, tm), tm), :]
    dsel2 = no_self * (d * d).astype(jnp.bfloat16).astype(jnp.float32)
    return acc + dsel2 * vrows.astype(jnp.float32)


def _layer1_kernel(c_ref, dall_ref, z_ref, b1_ref, w2_ref, o_ref, *, tm):
    h = _scaled_dot(c_ref, dall_ref, z_ref, tm) + b1_ref[...]
    h = jnp.where(h > 0, h, 0.2 * h)
    o_ref[...] = jnp.dot(h.astype(jnp.bfloat16), w2_ref[...],
                         preferred_element_type=jnp.float32
                         ).astype(jnp.bfloat16)


def _layer2_kernel(c_ref, dall_ref, u_ref, b2_ref, o_ref, *, tm):
    y = _scaled_dot(c_ref, dall_ref, u_ref, tm) + b2_ref[...]
    m = jnp.max(y, axis=1, keepdims=True)
    e = jnp.exp(y - m)
    o_ref[...] = y - (m + jnp.log(jnp.sum(e, axis=1, keepdims=True)))


def kernel(x, edge_index, w1, b1, w2, b2):
    n, fin = x.shape
    hidden = w1.shape[1]
    c = w2.shape[1]
    tm = 512 if n % 512 == 0 else n
    grid = (n // tm,)

    src = edge_index[0].astype(jnp.int32)
    dst = edge_index[1].astype(jnp.int32)
    counts = jnp.zeros((n, n), jnp.bfloat16).at[dst, src].add(
        jnp.ones(src.shape, jnp.bfloat16))

    w1b = w1.astype(jnp.bfloat16)
    w2b = w2.astype(jnp.bfloat16)
    b1f = b1.reshape(1, hidden).astype(jnp.float32)
    b2f = b2.reshape(1, c).astype(jnp.float32)

    params = pltpu.CompilerParams(
        dimension_semantics=("parallel",),
        vmem_limit_bytes=64 << 20,
    )

    z, dall = pl.pallas_call(
        functools.partial(_prep_kernel, tm=tm),
        out_shape=(jax.ShapeDtypeStruct((n, hidden), jnp.bfloat16),
                   jax.ShapeDtypeStruct((1, n), jnp.float32)),
        grid=grid,
        in_specs=[pl.BlockSpec((tm, fin), lambda i: (i, 0)),
                  pl.BlockSpec((fin, hidden), lambda i: (0, 0)),
                  pl.BlockSpec((tm, n), lambda i: (i, 0))],
        out_specs=(pl.BlockSpec((tm, hidden), lambda i: (i, 0)),
                   pl.BlockSpec((1, tm), lambda i: (0, i))),
        compiler_params=params,
        cost_estimate=pl.CostEstimate(
            flops=2 * n * fin * hidden, transcendentals=n,
            bytes_accessed=n * fin * 4 + n * n * 2 + n * hidden * 2),
    )(x, w1b, counts)

    u = pl.pallas_call(
        functools.partial(_layer1_kernel, tm=tm),
        out_shape=jax.ShapeDtypeStruct((n, c), jnp.bfloat16),
        grid=grid,
        in_specs=[pl.BlockSpec((tm, n), lambda i: (i, 0)),
                  pl.BlockSpec((1, n), lambda i: (0, 0)),
                  pl.BlockSpec((n, hidden), lambda i: (0, 0)),
                  pl.BlockSpec((1, hidden), lambda i: (0, 0)),
                  pl.BlockSpec((hidden, c), lambda i: (0, 0))],
        out_specs=pl.BlockSpec((tm, c), lambda i: (i, 0)),
        compiler_params=params,
        cost_estimate=pl.CostEstimate(
            flops=2 * n * n * hidden + 2 * n * hidden * c, transcendentals=n,
            bytes_accessed=n * n * 2 + n * hidden * 2 + n * c * 2),
    )(counts, dall, z, b1f, w2b)

    out = pl.pallas_call(
        functools.partial(_layer2_kernel, tm=tm),
        out_shape=jax.ShapeDtypeStruct((n, c), jnp.float32),
        grid=grid,
        in_specs=[pl.BlockSpec((tm, n), lambda i: (i, 0)),
                  pl.BlockSpec((1, n), lambda i: (0, 0)),
                  pl.BlockSpec((n, c), lambda i: (0, 0)),
                  pl.BlockSpec((1, c), lambda i: (0, 0))],
        out_specs=pl.BlockSpec((tm, c), lambda i: (i, 0)),
        compiler_params=params,
        cost_estimate=pl.CostEstimate(
            flops=2 * n * n * c, transcendentals=n * c + 2 * n,
            bytes_accessed=n * n * 2 + n * c * 2 + n * c * 4),
    )(counts, dall, u, b2f)

    return out


# speedup vs baseline: 4.9718x; 4.3852x over previous
import functools

import jax
import jax.numpy as jnp
from jax.experimental import pallas as pl
from jax.experimental.pallas import tpu as pltpu


def _tile_norm(c_ref, tm):
    i = pl.program_id(0)
    deg = jnp.sum(c_ref[...].astype(jnp.float32), axis=1, keepdims=True)
    csub = c_ref[:, pl.ds(pl.multiple_of(i * tm, tm), tm)].astype(jnp.float32)
    r = jax.lax.broadcasted_iota(jnp.int32, (tm, tm), 0)
    col = jax.lax.broadcasted_iota(jnp.int32, (tm, tm), 1)
    self_cnt = jnp.sum(jnp.where(r == col, csub, 0.0), axis=1, keepdims=True)
    no_self = (self_cnt == 0.0).astype(jnp.float32)
    d = jax.lax.rsqrt(deg + no_self)
    return d, no_self


def _prep_kernel(x_ref, w1_ref, c_ref, z_ref, d_ref, cb_ref, *, tm):
    d, _ = _tile_norm(c_ref, tm)
    z_ref[...] = jnp.dot(x_ref[...].astype(jnp.bfloat16), w1_ref[...],
                         preferred_element_type=jnp.float32
                         ).astype(jnp.bfloat16)
    d_ref[...] = d.reshape(1, tm)
    cb_ref[...] = c_ref[...].astype(jnp.bfloat16)


def _scaled_dot(c_ref, dall_ref, v_ref, tm):
    i = pl.program_id(0)
    d, no_self = _tile_norm(c_ref, tm)
    a = ((c_ref[...].astype(jnp.float32) * d) * dall_ref[...]
         ).astype(jnp.bfloat16)
    acc = jnp.dot(a, v_ref[...], preferred_element_type=jnp.float32)
    vrows = v_ref[pl.ds(pl.multiple_of(i * tm, tm), tm), :]
    dsel2 = no_self * (d * d).astype(jnp.bfloat16).astype(jnp.float32)
    return acc + dsel2 * vrows.astype(jnp.float32)


def _layer1_kernel(c_ref, dall_ref, z_ref, b1_ref, w2_ref, o_ref, *, tm):
    h = _scaled_dot(c_ref, dall_ref, z_ref, tm) + b1_ref[...]
    h = jnp.where(h > 0, h, 0.2 * h)
    o_ref[...] = jnp.dot(h.astype(jnp.bfloat16), w2_ref[...],
                         preferred_element_type=jnp.float32
                         ).astype(jnp.bfloat16)


def _layer2_kernel(c_ref, dall_ref, u_ref, b2_ref, o_ref, *, tm):
    y = _scaled_dot(c_ref, dall_ref, u_ref, tm) + b2_ref[...]
    m = jnp.max(y, axis=1, keepdims=True)
    e = jnp.exp(y - m)
    o_ref[...] = y - (m + jnp.log(jnp.sum(e, axis=1, keepdims=True)))


def kernel(x, edge_index, w1, b1, w2, b2):
    n, fin = x.shape
    hidden = w1.shape[1]
    c = w2.shape[1]
    tm = 512 if n % 512 == 0 else n
    grid = (n // tm,)

    src = edge_index[0].astype(jnp.int32)
    dst = edge_index[1].astype(jnp.int32)
    counts = jnp.zeros((n, n), jnp.float32).at[dst, src].add(1.0)

    w1b = w1.astype(jnp.bfloat16)
    w2b = w2.astype(jnp.bfloat16)
    b1f = b1.reshape(1, hidden).astype(jnp.float32)
    b2f = b2.reshape(1, c).astype(jnp.float32)

    params = pltpu.CompilerParams(
        dimension_semantics=("parallel",),
        vmem_limit_bytes=64 << 20,
    )

    z, dall, cb = pl.pallas_call(
        functools.partial(_prep_kernel, tm=tm),
        out_shape=(jax.ShapeDtypeStruct((n, hidden), jnp.bfloat16),
                   jax.ShapeDtypeStruct((1, n), jnp.float32),
                   jax.ShapeDtypeStruct((n, n), jnp.bfloat16)),
        grid=grid,
        in_specs=[pl.BlockSpec((tm, fin), lambda i: (i, 0)),
                  pl.BlockSpec((fin, hidden), lambda i: (0, 0)),
                  pl.BlockSpec((tm, n), lambda i: (i, 0))],
        out_specs=(pl.BlockSpec((tm, hidden), lambda i: (i, 0)),
                   pl.BlockSpec((1, tm), lambda i: (0, i)),
                   pl.BlockSpec((tm, n), lambda i: (i, 0))),
        compiler_params=params,
        cost_estimate=pl.CostEstimate(
            flops=2 * n * fin * hidden, transcendentals=n,
            bytes_accessed=n * fin * 4 + n * n * 6 + n * hidden * 2),
    )(x, w1b, counts)

    u = pl.pallas_call(
        functools.partial(_layer1_kernel, tm=tm),
        out_shape=jax.ShapeDtypeStruct((n, c), jnp.bfloat16),
        grid=grid,
        in_specs=[pl.BlockSpec((tm, n), lambda i: (i, 0)),
                  pl.BlockSpec((1, n), lambda i: (0, 0)),
                  pl.BlockSpec((n, hidden), lambda i: (0, 0)),
                  pl.BlockSpec((1, hidden), lambda i: (0, 0)),
                  pl.BlockSpec((hidden, c), lambda i: (0, 0))],
        out_specs=pl.BlockSpec((tm, c), lambda i: (i, 0)),
        compiler_params=params,
        cost_estimate=pl.CostEstimate(
            flops=2 * n * n * hidden + 2 * n * hidden * c, transcendentals=n,
            bytes_accessed=n * n * 2 + n * hidden * 2 + n * c * 2),
    )(cb, dall, z, b1f, w2b)

    out = pl.pallas_call(
        functools.partial(_layer2_kernel, tm=tm),
        out_shape=jax.ShapeDtypeStruct((n, c), jnp.float32),
        grid=grid,
        in_specs=[pl.BlockSpec((tm, n), lambda i: (i, 0)),
                  pl.BlockSpec((1, n), lambda i: (0, 0)),
                  pl.BlockSpec((n, c), lambda i: (0, 0)),
                  pl.BlockSpec((1, c), lambda i: (0, 0))],
        out_specs=pl.BlockSpec((tm, c), lambda i: (i, 0)),
        compiler_params=params,
        cost_estimate=pl.CostEstimate(
            flops=2 * n * n * c, transcendentals=n * c + 2 * n,
            bytes_accessed=n * n * 2 + n * c * 2 + n * c * 4),
    )(cb, dall, u, b2f)

    return out


# speedup vs baseline: 6.4447x; 1.2962x over previous
import functools

import jax
import jax.numpy as jnp
from jax.experimental import pallas as pl
from jax.experimental.pallas import tpu as pltpu

_HI = 65536.0


def _tile_norm(c_ref, tm):
    i = pl.program_id(0)
    deg = jnp.sum(c_ref[...].astype(jnp.float32), axis=1, keepdims=True)
    csub = c_ref[:, pl.ds(pl.multiple_of(i * tm, tm), tm)].astype(jnp.float32)
    r = jax.lax.broadcasted_iota(jnp.int32, (tm, tm), 0)
    col = jax.lax.broadcasted_iota(jnp.int32, (tm, tm), 1)
    self_cnt = jnp.sum(jnp.where(r == col, csub, 0.0), axis=1, keepdims=True)
    no_self = (self_cnt == 0.0).astype(jnp.float32)
    d = jax.lax.rsqrt(deg + no_self)
    return d, no_self


def _xw_kernel(x_ref, w1_ref, o_ref):
    o_ref[...] = jnp.dot(x_ref[...].astype(jnp.bfloat16), w1_ref[...],
                         preferred_element_type=jnp.float32
                         ).astype(jnp.bfloat16)


def _unpack_kernel(p_ref, cb_ref, d_ref, *, tm, ncol):
    i = pl.program_id(0)
    p = p_ref[...]
    high = jnp.floor(p * (1.0 / _HI))
    low = p - high * _HI
    cb_ref[:, :ncol] = low.astype(jnp.bfloat16)
    cb_ref[:, ncol:] = high.astype(jnp.bfloat16)
    deg = (jnp.sum(low, axis=1, keepdims=True)
           + jnp.sum(high, axis=1, keepdims=True))
    half = (i * tm) // ncol
    pc = (i * tm) % ncol
    psub = p_ref[:, pl.ds(pl.multiple_of(pc, tm), tm)]
    hsub = jnp.floor(psub * (1.0 / _HI))
    fsub = jnp.where(half == 1, hsub, psub - hsub * _HI)
    r = jax.lax.broadcasted_iota(jnp.int32, (tm, tm), 0)
    col = jax.lax.broadcasted_iota(jnp.int32, (tm, tm), 1)
    self_cnt = jnp.sum(jnp.where(r == col, fsub, 0.0), axis=1, keepdims=True)
    no_self = (self_cnt == 0.0).astype(jnp.float32)
    d = jax.lax.rsqrt(deg + no_self)
    d_ref[...] = d.reshape(1, tm)


def _scaled_dot(c_ref, dall_ref, v_ref, tm):
    i = pl.program_id(0)
    d, no_self = _tile_norm(c_ref, tm)
    a = ((c_ref[...].astype(jnp.float32) * d) * dall_ref[...]
         ).astype(jnp.bfloat16)
    acc = jnp.dot(a, v_ref[...], preferred_element_type=jnp.float32)
    vrows = v_ref[pl.ds(pl.multiple_of(i * tm, tm), tm), :]
    dsel2 = no_self * (d * d).astype(jnp.bfloat16).astype(jnp.float32)
    return acc + dsel2 * vrows.astype(jnp.float32)


def _layer1_kernel(c_ref, dall_ref, z_ref, b1_ref, w2_ref, o_ref, *, tm):
    h = _scaled_dot(c_ref, dall_ref, z_ref, tm) + b1_ref[...]
    h = jnp.where(h > 0, h, 0.2 * h)
    o_ref[...] = jnp.dot(h.astype(jnp.bfloat16), w2_ref[...],
                         preferred_element_type=jnp.float32
                         ).astype(jnp.bfloat16)


def _layer2_kernel(c_ref, dall_ref, u_ref, b2_ref, o_ref, *, tm):
    y = _scaled_dot(c_ref, dall_ref, u_ref, tm) + b2_ref[...]
    m = jnp.max(y, axis=1, keepdims=True)
    e = jnp.exp(y - m)
    o_ref[...] = y - (m + jnp.log(jnp.sum(e, axis=1, keepdims=True)))


def kernel(x, edge_index, w1, b1, w2, b2):
    n, fin = x.shape
    hidden = w1.shape[1]
    c = w2.shape[1]
    tm = 512 if n % 512 == 0 else n
    grid = (n // tm,)
    ncol = n // 2

    src = edge_index[0].astype(jnp.int32)
    dst = edge_index[1].astype(jnp.int32)
    col = jnp.where(src >= ncol, src - ncol, src)
    val = jnp.where(src >= ncol, _HI, 1.0).astype(jnp.float32)
    packed = jnp.zeros((n, ncol), jnp.float32).at[dst, col].add(val)

    w1b = w1.astype(jnp.bfloat16)
    w2b = w2.astype(jnp.bfloat16)
    b1f = b1.reshape(1, hidden).astype(jnp.float32)
    b2f = b2.reshape(1, c).astype(jnp.float32)

    params = pltpu.CompilerParams(
        dimension_semantics=("parallel",),
        vmem_limit_bytes=64 << 20,
    )

    z = pl.pallas_call(
        _xw_kernel,
        out_shape=jax.ShapeDtypeStruct((n, hidden), jnp.bfloat16),
        grid=grid,
        in_specs=[pl.BlockSpec((tm, fin), lambda i: (i, 0)),
                  pl.BlockSpec((fin, hidden), lambda i: (0, 0))],
        out_specs=pl.BlockSpec((tm, hidden), lambda i: (i, 0)),
        compiler_params=params,
        cost_estimate=pl.CostEstimate(
            flops=2 * n * fin * hidden, transcendentals=0,
            bytes_accessed=n * fin * 4 + fin * hidden * 2 + n * hidden * 2),
    )(x, w1b)

    cb, dall = pl.pallas_call(
        functools.partial(_unpack_kernel, tm=tm, ncol=ncol),
        out_shape=(jax.ShapeDtypeStruct((n, n), jnp.bfloat16),
                   jax.ShapeDtypeStruct((1, n), jnp.float32)),
        grid=grid,
        in_specs=[pl.BlockSpec((tm, ncol), lambda i: (i, 0))],
        out_specs=(pl.BlockSpec((tm, n), lambda i: (i, 0)),
                   pl.BlockSpec((1, tm), lambda i: (0, i))),
        compiler_params=params,
        cost_estimate=pl.CostEstimate(
            flops=0, transcendentals=n,
            bytes_accessed=n * ncol * 4 + n * n * 2),
    )(packed)

    u = pl.pallas_call(
        functools.partial(_layer1_kernel, tm=tm),
        out_shape=jax.ShapeDtypeStruct((n, c), jnp.bfloat16),
        grid=grid,
        in_specs=[pl.BlockSpec((tm, n), lambda i: (i, 0)),
                  pl.BlockSpec((1, n), lambda i: (0, 0)),
                  pl.BlockSpec((n, hidden), lambda i: (0, 0)),
                  pl.BlockSpec((1, hidden), lambda i: (0, 0)),
                  pl.BlockSpec((hidden, c), lambda i: (0, 0))],
        out_specs=pl.BlockSpec((tm, c), lambda i: (i, 0)),
        compiler_params=params,
        cost_estimate=pl.CostEstimate(
            flops=2 * n * n * hidden + 2 * n * hidden * c, transcendentals=n,
            bytes_accessed=n * n * 2 + n * hidden * 2 + n * c * 2),
    )(cb, dall, z, b1f, w2b)

    out = pl.pallas_call(
        functools.partial(_layer2_kernel, tm=tm),
        out_shape=jax.ShapeDtypeStruct((n, c), jnp.float32),
        grid=grid,
        in_specs=[pl.BlockSpec((tm, n), lambda i: (i, 0)),
                  pl.BlockSpec((1, n), lambda i: (0, 0)),
                  pl.BlockSpec((n, c), lambda i: (0, 0)),
                  pl.BlockSpec((1, c), lambda i: (0, 0))],
        out_specs=pl.BlockSpec((tm, c), lambda i: (i, 0)),
        compiler_params=params,
        cost_estimate=pl.CostEstimate(
            flops=2 * n * n * c, transcendentals=n * c + 2 * n,
            bytes_accessed=n * n * 2 + n * c * 2 + n * c * 4),
    )(cb, dall, u, b2f)

    return out


# speedup vs baseline: 6.5860x; 1.0219x over previous
import functools

import jax
import jax.numpy as jnp
from jax.experimental import pallas as pl
from jax.experimental.pallas import tpu as pltpu

_HI = 65536.0


def _norm_from_packed(p_ref, tm, ncol):
    i = pl.program_id(0)
    p = p_ref[...]
    hi = jnp.floor(p * (1.0 / _HI))
    lo = p - hi * _HI
    deg = (jnp.sum(lo, axis=1, keepdims=True)
           + jnp.sum(hi, axis=1, keepdims=True))
    half = (i * tm) // ncol
    pc = (i * tm) % ncol
    psub = p_ref[:, pl.ds(pl.multiple_of(pc, tm), tm)]
    hsub = jnp.floor(psub * (1.0 / _HI))
    fsub = jnp.where(half == 1, hsub, psub - hsub * _HI)
    r = jax.lax.broadcasted_iota(jnp.int32, (tm, tm), 0)
    col = jax.lax.broadcasted_iota(jnp.int32, (tm, tm), 1)
    self_cnt = jnp.sum(jnp.where(r == col, fsub, 0.0), axis=1, keepdims=True)
    no_self = (self_cnt == 0.0).astype(jnp.float32)
    d = jax.lax.rsqrt(deg + no_self)
    return lo, hi, d, no_self


def _xw_kernel(x_ref, w1_ref, o_ref):
    o_ref[...] = jnp.dot(x_ref[...].astype(jnp.bfloat16), w1_ref[...],
                         preferred_element_type=jnp.float32
                         ).astype(jnp.bfloat16)


def _dvec_kernel(p_ref, d_ref, *, tm, ncol):
    _, _, d, _ = _norm_from_packed(p_ref, tm, ncol)
    d_ref[...] = d.reshape(1, tm)


def _scaled_dot(p_ref, dall_ref, v_ref, tm, ncol):
    i = pl.program_id(0)
    lo, hi, d, no_self = _norm_from_packed(p_ref, tm, ncol)
    a0 = ((lo * d) * dall_ref[:, :ncol]).astype(jnp.bfloat16)
    a1 = ((hi * d) * dall_ref[:, ncol:]).astype(jnp.bfloat16)
    v0 = v_ref[pl.ds(0, ncol), :]
    v1 = v_ref[pl.ds(ncol, ncol), :]
    acc = (jnp.dot(a0, v0, preferred_element_type=jnp.float32)
           + jnp.dot(a1, v1, preferred_element_type=jnp.float32))
    vrows = v_ref[pl.ds(pl.multiple_of(i * tm, tm), tm), :]
    dsel2 = no_self * (d * d).astype(jnp.bfloat16).astype(jnp.float32)
    return acc + dsel2 * vrows.astype(jnp.float32)


def _layer1_kernel(p_ref, dall_ref, z_ref, b1_ref, w2_ref, o_ref, *, tm,
                   ncol):
    h = _scaled_dot(p_ref, dall_ref, z_ref, tm, ncol) + b1_ref[...]
    h = jnp.where(h > 0, h, 0.2 * h)
    o_ref[...] = jnp.dot(h.astype(jnp.bfloat16), w2_ref[...],
                         preferred_element_type=jnp.float32
                         ).astype(jnp.bfloat16)


def _layer2_kernel(p_ref, dall_ref, u_ref, b2_ref, o_ref, *, tm, ncol):
    y = _scaled_dot(p_ref, dall_ref, u_ref, tm, ncol) + b2_ref[...]
    m = jnp.max(y, axis=1, keepdims=True)
    e = jnp.exp(y - m)
    o_ref[...] = y - (m + jnp.log(jnp.sum(e, axis=1, keepdims=True)))


def kernel(x, edge_index, w1, b1, w2, b2):
    n, fin = x.shape
    hidden = w1.shape[1]
    c = w2.shape[1]
    tm = 512 if n % 512 == 0 else n
    grid = (n // tm,)
    ncol = n // 2

    src = edge_index[0].astype(jnp.int32)
    dst = edge_index[1].astype(jnp.int32)
    col = jnp.where(src >= ncol, src - ncol, src)
    val = jnp.where(src >= ncol, _HI, 1.0).astype(jnp.float32)
    packed = jnp.zeros((n, ncol), jnp.float32).at[dst, col].add(val)

    w1b = w1.astype(jnp.bfloat16)
    w2b = w2.astype(jnp.bfloat16)
    b1f = b1.reshape(1, hidden).astype(jnp.float32)
    b2f = b2.reshape(1, c).astype(jnp.float32)

    params = pltpu.CompilerParams(
        dimension_semantics=("parallel",),
        vmem_limit_bytes=64 << 20,
    )

    z = pl.pallas_call(
        _xw_kernel,
        out_shape=jax.ShapeDtypeStruct((n, hidden), jnp.bfloat16),
        grid=grid,
        in_specs=[pl.BlockSpec((tm, fin), lambda i: (i, 0)),
                  pl.BlockSpec((fin, hidden), lambda i: (0, 0))],
        out_specs=pl.BlockSpec((tm, hidden), lambda i: (i, 0)),
        compiler_params=params,
        cost_estimate=pl.CostEstimate(
            flops=2 * n * fin * hidden, transcendentals=0,
            bytes_accessed=n * fin * 4 + fin * hidden * 2 + n * hidden * 2),
    )(x, w1b)

    dall = pl.pallas_call(
        functools.partial(_dvec_kernel, tm=tm, ncol=ncol),
        out_shape=jax.ShapeDtypeStruct((1, n), jnp.float32),
        grid=grid,
        in_specs=[pl.BlockSpec((tm, ncol), lambda i: (i, 0))],
        out_specs=pl.BlockSpec((1, tm), lambda i: (0, i)),
        compiler_params=params,
        cost_estimate=pl.CostEstimate(
            flops=0, transcendentals=n,
            bytes_accessed=n * ncol * 4 + n * 4),
    )(packed)

    u = pl.pallas_call(
        functools.partial(_layer1_kernel, tm=tm, ncol=ncol),
        out_shape=jax.ShapeDtypeStruct((n, c), jnp.bfloat16),
        grid=grid,
        in_specs=[pl.BlockSpec((tm, ncol), lambda i: (i, 0)),
                  pl.BlockSpec((1, n), lambda i: (0, 0)),
                  pl.BlockSpec((n, hidden), lambda i: (0, 0)),
                  pl.BlockSpec((1, hidden), lambda i: (0, 0)),
                  pl.BlockSpec((hidden, c), lambda i: (0, 0))],
        out_specs=pl.BlockSpec((tm, c), lambda i: (i, 0)),
        compiler_params=params,
        cost_estimate=pl.CostEstimate(
            flops=2 * n * n * hidden + 2 * n * hidden * c, transcendentals=n,
            bytes_accessed=n * ncol * 4 + n * hidden * 2 + n * c * 2),
    )(packed, dall, z, b1f, w2b)

    out = pl.pallas_call(
        functools.partial(_layer2_kernel, tm=tm, ncol=ncol),
        out_shape=jax.ShapeDtypeStruct((n, c), jnp.float32),
        grid=grid,
        in_specs=[pl.BlockSpec((tm, ncol), lambda i: (i, 0)),
                  pl.BlockSpec((1, n), lambda i: (0, 0)),
                  pl.BlockSpec((n, c), lambda i: (0, 0)),
                  pl.BlockSpec((1, c), lambda i: (0, 0))],
        out_specs=pl.BlockSpec((tm, c), lambda i: (i, 0)),
        compiler_params=params,
        cost_estimate=pl.CostEstimate(
            flops=2 * n * n * c, transcendentals=n * c + 2 * n,
            bytes_accessed=n * ncol * 4 + n * c * 2 + n * c * 4),
    )(packed, dall, u, b2f)

    return out


# speedup vs baseline: 7.2653x; 1.1031x over previous
import functools

import jax
import jax.numpy as jnp
from jax.experimental import pallas as pl
from jax.experimental.pallas import tpu as pltpu

_B = 64.0


def _unpack4(p):
    f3 = jnp.floor(p * (1.0 / _B ** 3))
    r3 = p - f3 * _B ** 3
    f2 = jnp.floor(r3 * (1.0 / _B ** 2))
    r2 = r3 - f2 * _B ** 2
    f1 = jnp.floor(r2 * (1.0 / _B))
    f0 = r2 - f1 * _B
    return f0, f1, f2, f3


def _norm_from_packed(p_ref, tm, nq):
    i = pl.program_id(0)
    f = _unpack4(p_ref[...])
    deg = (jnp.sum(f[0] + f[1], axis=1, keepdims=True)
           + jnp.sum(f[2] + f[3], axis=1, keepdims=True))
    dig = (i * tm) // nq
    pc = (i * tm) % nq
    psub = p_ref[:, pl.ds(pl.multiple_of(pc, tm), tm)]
    v = jnp.floor(psub * jnp.exp2(-6.0 * dig.astype(jnp.float32)))
    fsub = v - jnp.floor(v * (1.0 / _B)) * _B
    r = jax.lax.broadcasted_iota(jnp.int32, (tm, tm), 0)
    col = jax.lax.broadcasted_iota(jnp.int32, (tm, tm), 1)
    self_cnt = jnp.sum(jnp.where(r == col, fsub, 0.0), axis=1, keepdims=True)
    no_self = (self_cnt == 0.0).astype(jnp.float32)
    d = jax.lax.rsqrt(deg + no_self)
    return f, d, no_self


def _prep_kernel(x_ref, w1_ref, p_ref, z_ref, d_ref, *, tm, nq):
    _, d, _ = _norm_from_packed(p_ref, tm, nq)
    z_ref[...] = jnp.dot(x_ref[...].astype(jnp.bfloat16), w1_ref[...],
                         preferred_element_type=jnp.float32
                         ).astype(jnp.bfloat16)
    d_ref[...] = d.reshape(1, tm)


def _scaled_dot(p_ref, dall_ref, v_ref, tm, nq):
    i = pl.program_id(0)
    f, d, no_self = _norm_from_packed(p_ref, tm, nq)
    acc = None
    for k in range(4):
        a_k = ((f[k] * d) * dall_ref[:, pl.ds(k * nq, nq)]
               ).astype(jnp.bfloat16)
        v_k = v_ref[pl.ds(k * nq, nq), :]
        t = jnp.dot(a_k, v_k, preferred_element_type=jnp.float32)
        acc = t if acc is None else acc + t
    vrows = v_ref[pl.ds(pl.multiple_of(i * tm, tm), tm), :]
    dsel2 = no_self * (d * d).astype(jnp.bfloat16).astype(jnp.float32)
    return acc + dsel2 * vrows.astype(jnp.float32)


def _layer1_kernel(p_ref, dall_ref, z_ref, b1_ref, w2_ref, o_ref, *, tm, nq):
    h = _scaled_dot(p_ref, dall_ref, z_ref, tm, nq) + b1_ref[...]
    h = jnp.where(h > 0, h, 0.2 * h)
    o_ref[...] = jnp.dot(h.astype(jnp.bfloat16), w2_ref[...],
                         preferred_element_type=jnp.float32
                         ).astype(jnp.bfloat16)


def _layer2_kernel(p_ref, dall_ref, u_ref, b2_ref, o_ref, *, tm, nq):
    y = _scaled_dot(p_ref, dall_ref, u_ref, tm, nq) + b2_ref[...]
    m = jnp.max(y, axis=1, keepdims=True)
    e = jnp.exp(y - m)
    o_ref[...] = y - (m + jnp.log(jnp.sum(e, axis=1, keepdims=True)))


def kernel(x, edge_index, w1, b1, w2, b2):
    n, fin = x.shape
    hidden = w1.shape[1]
    c = w2.shape[1]
    nq = n // 4
    tm = min(512, nq)
    grid = (n // tm,)

    src = edge_index[0].astype(jnp.int32)
    dst = edge_index[1].astype(jnp.int32)
    dig = src // nq
    col = src - dig * nq
    val = jnp.where(dig == 0, 1.0,
                    jnp.where(dig == 1, _B,
                              jnp.where(dig == 2, _B ** 2, _B ** 3)))
    packed = jnp.zeros((n, nq), jnp.float32).at[dst, col].add(
        val.astype(jnp.float32))

    w1b = w1.astype(jnp.bfloat16)
    w2b = w2.astype(jnp.bfloat16)
    b1f = b1.reshape(1, hidden).astype(jnp.float32)
    b2f = b2.reshape(1, c).astype(jnp.float32)

    params = pltpu.CompilerParams(
        dimension_semantics=("parallel",),
        vmem_limit_bytes=64 << 20,
    )

    z, dall = pl.pallas_call(
        functools.partial(_prep_kernel, tm=tm, nq=nq),
        out_shape=(jax.ShapeDtypeStruct((n, hidden), jnp.bfloat16),
                   jax.ShapeDtypeStruct((1, n), jnp.float32)),
        grid=grid,
        in_specs=[pl.BlockSpec((tm, fin), lambda i: (i, 0)),
                  pl.BlockSpec((fin, hidden), lambda i: (0, 0)),
                  pl.BlockSpec((tm, nq), lambda i: (i, 0))],
        out_specs=(pl.BlockSpec((tm, hidden), lambda i: (i, 0)),
                   pl.BlockSpec((1, tm), lambda i: (0, i))),
        compiler_params=params,
        cost_estimate=pl.CostEstimate(
            flops=2 * n * fin * hidden, transcendentals=n,
            bytes_accessed=n * fin * 4 + n * nq * 4 + n * hidden * 2),
    )(x, w1b, packed)

    u = pl.pallas_call(
        functools.partial(_layer1_kernel, tm=tm, nq=nq),
        out_shape=jax.ShapeDtypeStruct((n, c), jnp.bfloat16),
        grid=grid,
        in_specs=[pl.BlockSpec((tm, nq), lambda i: (i, 0)),
                  pl.BlockSpec((1, n), lambda i: (0, 0)),
                  pl.BlockSpec((n, hidden), lambda i: (0, 0)),
                  pl.BlockSpec((1, hidden), lambda i: (0, 0)),
                  pl.BlockSpec((hidden, c), lambda i: (0, 0))],
        out_specs=pl.BlockSpec((tm, c), lambda i: (i, 0)),
        compiler_params=params,
        cost_estimate=pl.CostEstimate(
            flops=2 * n * n * hidden + 2 * n * hidden * c, transcendentals=n,
            bytes_accessed=n * nq * 4 + n * hidden * 2 + n * c * 2),
    )(packed, dall, z, b1f, w2b)

    out = pl.pallas_call(
        functools.partial(_layer2_kernel, tm=tm, nq=nq),
        out_shape=jax.ShapeDtypeStruct((n, c), jnp.float32),
        grid=grid,
        in_specs=[pl.BlockSpec((tm, nq), lambda i: (i, 0)),
                  pl.BlockSpec((1, n), lambda i: (0, 0)),
                  pl.BlockSpec((n, c), lambda i: (0, 0)),
                  pl.BlockSpec((1, c), lambda i: (0, 0))],
        out_specs=pl.BlockSpec((tm, c), lambda i: (i, 0)),
        compiler_params=params,
        cost_estimate=pl.CostEstimate(
            flops=2 * n * n * c, transcendentals=n * c + 2 * n,
            bytes_accessed=n * nq * 4 + n * c * 2 + n * c * 4),
    )(packed, dall, u, b2f)

    return out


# speedup vs baseline: 7.3996x; 1.0185x over previous
import functools

import jax
import jax.numpy as jnp
from jax.experimental import pallas as pl
from jax.experimental.pallas import tpu as pltpu

_B = 64.0


def _unpack4(p):
    f3 = jnp.floor(p * (1.0 / _B ** 3))
    r3 = p - f3 * _B ** 3
    f2 = jnp.floor(r3 * (1.0 / _B ** 2))
    r2 = r3 - f2 * _B ** 2
    f1 = jnp.floor(r2 * (1.0 / _B))
    f0 = r2 - f1 * _B
    return f0, f1, f2, f3


def _norm_from_packed(p_ref, tm, nq):
    i = pl.program_id(0)
    f = _unpack4(p_ref[...])
    deg = (jnp.sum(f[0] + f[1], axis=1, keepdims=True)
           + jnp.sum(f[2] + f[3], axis=1, keepdims=True))
    dig = (i * tm) // nq
    pc = (i * tm) % nq
    psub = p_ref[:, pl.ds(pl.multiple_of(pc, tm), tm)]
    v = jnp.floor(psub * jnp.exp2(-6.0 * dig.astype(jnp.float32)))
    fsub = v - jnp.floor(v * (1.0 / _B)) * _B
    r = jax.lax.broadcasted_iota(jnp.int32, (tm, tm), 0)
    col = jax.lax.broadcasted_iota(jnp.int32, (tm, tm), 1)
    self_cnt = jnp.sum(jnp.where(r == col, fsub, 0.0), axis=1, keepdims=True)
    no_self = (self_cnt == 0.0).astype(jnp.float32)
    d = jax.lax.rsqrt(deg + no_self)
    return f, d, no_self


def _xw_kernel(x_ref, w1_ref, o_ref):
    o_ref[...] = jnp.dot(x_ref[...].astype(jnp.bfloat16), w1_ref[...],
                         preferred_element_type=jnp.float32
                         ).astype(jnp.bfloat16)


def _dvec_kernel(p_ref, d_ref, *, tm, nq):
    _, d, _ = _norm_from_packed(p_ref, tm, nq)
    d_ref[...] = d.reshape(1, tm)


def _scaled_dot(p_ref, dall_ref, v_ref, tm, nq):
    i = pl.program_id(0)
    f, d, no_self = _norm_from_packed(p_ref, tm, nq)
    acc = None
    for k in range(4):
        a_k = ((f[k] * d) * dall_ref[:, pl.ds(k * nq, nq)]
               ).astype(jnp.bfloat16)
        v_k = v_ref[pl.ds(k * nq, nq), :]
        t = jnp.dot(a_k, v_k, preferred_element_type=jnp.float32)
        acc = t if acc is None else acc + t
    vrows = v_ref[pl.ds(pl.multiple_of(i * tm, tm), tm), :]
    dsel2 = no_self * (d * d).astype(jnp.bfloat16).astype(jnp.float32)
    return acc + dsel2 * vrows.astype(jnp.float32)


def _layer1_kernel(p_ref, dall_ref, z_ref, b1_ref, w2_ref, o_ref, *, tm, nq):
    h = _scaled_dot(p_ref, dall_ref, z_ref, tm, nq) + b1_ref[...]
    h = jnp.where(h > 0, h, 0.2 * h)
    o_ref[...] = jnp.dot(h.astype(jnp.bfloat16), w2_ref[...],
                         preferred_element_type=jnp.float32
                         ).astype(jnp.bfloat16)


def _layer2_kernel(p_ref, dall_ref, u_ref, b2_ref, o_ref, *, tm, nq):
    y = _scaled_dot(p_ref, dall_ref, u_ref, tm, nq) + b2_ref[...]
    m = jnp.max(y, axis=1, keepdims=True)
    e = jnp.exp(y - m)
    o_ref[...] = y - (m + jnp.log(jnp.sum(e, axis=1, keepdims=True)))


def kernel(x, edge_index, w1, b1, w2, b2):
    n, fin = x.shape
    hidden = w1.shape[1]
    c = w2.shape[1]
    nq = n // 4
    tm = min(512, nq)
    grid = (n // tm,)

    src = edge_index[0].astype(jnp.int32)
    dst = edge_index[1].astype(jnp.int32)
    dig = src // nq
    col = src - dig * nq
    val = jnp.where(dig == 0, 1.0,
                    jnp.where(dig == 1, _B,
                              jnp.where(dig == 2, _B ** 2, _B ** 3)))
    packed = jnp.zeros((n, nq), jnp.float32).at[dst, col].add(
        val.astype(jnp.float32))

    w1b = w1.astype(jnp.bfloat16)
    w2b = w2.astype(jnp.bfloat16)
    b1f = b1.reshape(1, hidden).astype(jnp.float32)
    b2f = b2.reshape(1, c).astype(jnp.float32)

    params = pltpu.CompilerParams(
        dimension_semantics=("parallel",),
        vmem_limit_bytes=64 << 20,
    )

    z = pl.pallas_call(
        _xw_kernel,
        out_shape=jax.ShapeDtypeStruct((n, hidden), jnp.bfloat16),
        grid=grid,
        in_specs=[pl.BlockSpec((tm, fin), lambda i: (i, 0)),
                  pl.BlockSpec((fin, hidden), lambda i: (0, 0))],
        out_specs=pl.BlockSpec((tm, hidden), lambda i: (i, 0)),
        compiler_params=params,
        cost_estimate=pl.CostEstimate(
            flops=2 * n * fin * hidden, transcendentals=0,
            bytes_accessed=n * fin * 4 + fin * hidden * 2 + n * hidden * 2),
    )(x, w1b)

    dall = pl.pallas_call(
        functools.partial(_dvec_kernel, tm=tm, nq=nq),
        out_shape=jax.ShapeDtypeStruct((1, n), jnp.float32),
        grid=grid,
        in_specs=[pl.BlockSpec((tm, nq), lambda i: (i, 0))],
        out_specs=pl.BlockSpec((1, tm), lambda i: (0, i)),
        compiler_params=params,
        cost_estimate=pl.CostEstimate(
            flops=0, transcendentals=n,
            bytes_accessed=n * nq * 4 + n * 4),
    )(packed)

    u = pl.pallas_call(
        functools.partial(_layer1_kernel, tm=tm, nq=nq),
        out_shape=jax.ShapeDtypeStruct((n, c), jnp.bfloat16),
        grid=grid,
        in_specs=[pl.BlockSpec((tm, nq), lambda i: (i, 0)),
                  pl.BlockSpec((1, n), lambda i: (0, 0)),
                  pl.BlockSpec((n, hidden), lambda i: (0, 0)),
                  pl.BlockSpec((1, hidden), lambda i: (0, 0)),
                  pl.BlockSpec((hidden, c), lambda i: (0, 0))],
        out_specs=pl.BlockSpec((tm, c), lambda i: (i, 0)),
        compiler_params=params,
        cost_estimate=pl.CostEstimate(
            flops=2 * n * n * hidden + 2 * n * hidden * c, transcendentals=n,
            bytes_accessed=n * nq * 4 + n * hidden * 2 + n * c * 2),
    )(packed, dall, z, b1f, w2b)

    out = pl.pallas_call(
        functools.partial(_layer2_kernel, tm=tm, nq=nq),
        out_shape=jax.ShapeDtypeStruct((n, c), jnp.float32),
        grid=grid,
        in_specs=[pl.BlockSpec((tm, nq), lambda i: (i, 0)),
                  pl.BlockSpec((1, n), lambda i: (0, 0)),
                  pl.BlockSpec((n, c), lambda i: (0, 0)),
                  pl.BlockSpec((1, c), lambda i: (0, 0))],
        out_specs=pl.BlockSpec((tm, c), lambda i: (i, 0)),
        compiler_params=params,
        cost_estimate=pl.CostEstimate(
            flops=2 * n * n * c, transcendentals=n * c + 2 * n,
            bytes_accessed=n * nq * 4 + n * c * 2 + n * c * 4),
    )(packed, dall, u, b2f)

    return out
